# baseline (speedup 1.0000x reference)
import functools

import jax
import jax.numpy as jnp
from jax import lax
from jax.experimental import pallas as pl
from jax.experimental.pallas import tpu as pltpu
from jax.experimental.pallas import tpu_sc as plsc

_D = 64
_HIST = 50
_L = 16
_CB = 128
_NBUF = 2
_NB = 1024


def _sc_geometry():
    try:
        info = plsc.get_sparse_core_info()
        return info.num_cores, info.num_subcores
    except Exception:
        return 2, 16


@functools.lru_cache(maxsize=None)
def _make_pooler(batch):
    nc, ns = _sc_geometry()
    nw = nc * ns
    bw = batch // nw
    nchunk_b = bw // _CB
    nchunk = _HIST * nchunk_b
    mesh = plsc.VectorSubcoreMesh(core_axis_name="c", subcore_axis_name="s")

    @functools.partial(
        pl.kernel,
        mesh=mesh,
        out_type=jax.ShapeDtypeStruct((batch, _D), jnp.float32),
        scratch_types=[
            pltpu.VMEM((_HIST, bw), jnp.int32),
            *[pltpu.VMEM((_CB, _D), jnp.float32) for _ in range(_NBUF)],
            pltpu.VMEM((bw, _D), jnp.float32),
            *[pltpu.SemaphoreType.DMA for _ in range(_NBUF)],
        ],
        compiler_params=pltpu.CompilerParams(use_tc_tiling_on_sc=False),
    )
    def pool(idx_hbm, table_hbm, out_hbm, idx_v, *rest):
        rows = rest[:_NBUF]
        acc_v = rest[_NBUF]
        sems = rest[_NBUF + 1:]
        w = lax.axis_index("s") * nc + lax.axis_index("c")
        base = w * bw
        pltpu.sync_copy(idx_hbm.at[:, pl.ds(base, bw)], idx_v)

        zeros = jnp.zeros((_L,), jnp.float32)

        def zbody(i, carry):
            for g in range(_D // _L):
                acc_v[i, pl.ds(g * _L, _L)] = zeros
            return carry

        lax.fori_loop(0, bw, zbody, 0)

        def idx_ref(c):
            j = c // nchunk_b
            b0 = (c % nchunk_b) * _CB
            return idx_v.at[j, pl.ds(b0, _CB)]

        def start(c, b):
            pltpu.async_copy(table_hbm.at[idx_ref(c)], rows[b], sems[b])

        def drain(b):
            pltpu.make_async_copy(table_hbm.at[idx_ref(0)], rows[b], sems[b]).wait()

        def accumulate(c, b):
            b0 = (c % nchunk_b) * _CB
            for r in range(_CB):
                for g in range(_D // _L):
                    col = pl.ds(g * _L, _L)
                    plsc.addupdate(acc_v.at[b0 + r, col], rows[b][r, col])

        for b in range(_NBUF):
            start(b, b)

        def group(gi, carry):
            c0 = gi * _NBUF
            for b in range(_NBUF):
                c = c0 + b
                drain(b)
                accumulate(c, b)

                @pl.when(c + _NBUF < nchunk)
                def _():
                    start(c + _NBUF, b)

            return carry

        lax.fori_loop(0, nchunk // _NBUF, group, 0)
        pltpu.sync_copy(acc_v, out_hbm.at[pl.ds(base, bw), :])

    return pool, nw


def _norm_kernel(x_ref, o_ref):
    x = x_ref[...]
    nsq = jnp.sum(x * x, axis=1, keepdims=True)
    clamp = jnp.float32((_HIST * 1e-12) ** 2)
    o_ref[...] = x * lax.rsqrt(jnp.maximum(nsq, clamp))


@functools.lru_cache(maxsize=None)
def _make_normalizer(batch):
    return pl.pallas_call(
        _norm_kernel,
        grid=(batch // _NB,),
        in_specs=[pl.BlockSpec((_NB, _D), lambda i: (i, 0))],
        out_specs=pl.BlockSpec((_NB, _D), lambda i: (i, 0)),
        out_shape=jax.ShapeDtypeStruct((batch, _D), jnp.float32),
    )


def kernel(indices, embeddings):
    b, h = indices.shape
    assert h == _HIST and embeddings.shape[1] == _D
    pool, nw = _make_pooler(b)
    idx_t = indices.astype(jnp.int32).T
    sums = pool(idx_t, embeddings.astype(jnp.float32))
    return _make_normalizer(b)(sums)


# speedup vs baseline: 1.4866x; 1.4866x over previous
import functools

import jax
import jax.numpy as jnp
from jax import lax
from jax.experimental import pallas as pl
from jax.experimental.pallas import tpu as pltpu
from jax.experimental.pallas import tpu_sc as plsc

_D = 64
_HIST = 50
_L = 16
_CB = 128
_NBUF = 2
_NB = 1024


def _sc_geometry():
    try:
        info = plsc.get_sparse_core_info()
        return info.num_cores, info.num_subcores
    except Exception:
        return 2, 16


@functools.lru_cache(maxsize=None)
def _make_pooler(batch):
    nc, ns = _sc_geometry()
    nw = nc * ns
    bw = batch // nw
    nchunk_b = bw // _CB
    nchunk = _HIST * nchunk_b
    mesh = plsc.VectorSubcoreMesh(core_axis_name="c", subcore_axis_name="s")

    @functools.partial(
        pl.kernel,
        mesh=mesh,
        out_type=jax.ShapeDtypeStruct((batch, _D), jnp.float32),
        scratch_types=[
            pltpu.VMEM((_HIST, bw), jnp.int32),
            pltpu.VMEM((bw, _D), jnp.float32),
            *[pltpu.SemaphoreType.DMA for _ in range(_NBUF)],
        ],
        compiler_params=pltpu.CompilerParams(use_tc_tiling_on_sc=False),
    )
    def pool(idx_hbm, table_hbm, out_hbm, idx_v, acc_v, *sems):
        w = lax.axis_index("s") * nc + lax.axis_index("c")
        base = w * bw
        pltpu.sync_copy(idx_hbm.at[:, pl.ds(base, bw)], idx_v)

        zeros = jnp.zeros((_L,), jnp.float32)

        def zbody(i, carry):
            for g in range(_D // _L):
                acc_v[i, pl.ds(g * _L, _L)] = zeros
            return carry

        lax.fori_loop(0, bw, zbody, 0)

        def idx_ref(c):
            j = c // nchunk_b
            b0 = (c % nchunk_b) * _CB
            return idx_v.at[j, pl.ds(b0, _CB)]

        def acc_ref(c):
            b0 = (c % nchunk_b) * _CB
            return acc_v.at[pl.ds(b0, _CB), :]

        def start(c, b):
            pltpu.async_copy(table_hbm.at[idx_ref(c)], acc_ref(c), sems[b], add=True)

        def drain(c, b):
            pltpu.make_async_copy(table_hbm.at[idx_ref(c)], acc_ref(c), sems[b]).wait()

        for b in range(_NBUF):
            start(b, b)

        def group(gi, carry):
            c0 = gi * _NBUF
            for b in range(_NBUF):
                c = c0 + b
                drain(c, b)

                @pl.when(c + _NBUF < nchunk)
                def _():
                    start(c + _NBUF, b)

            return carry

        lax.fori_loop(0, nchunk // _NBUF, group, 0)
        pltpu.sync_copy(acc_v, out_hbm.at[pl.ds(base, bw), :])

    return pool, nw


def _norm_kernel(x_ref, o_ref):
    x = x_ref[...]
    nsq = jnp.sum(x * x, axis=1, keepdims=True)
    clamp = jnp.float32((_HIST * 1e-12) ** 2)
    o_ref[...] = x * lax.rsqrt(jnp.maximum(nsq, clamp))


@functools.lru_cache(maxsize=None)
def _make_normalizer(batch):
    return pl.pallas_call(
        _norm_kernel,
        grid=(batch // _NB,),
        in_specs=[pl.BlockSpec((_NB, _D), lambda i: (i, 0))],
        out_specs=pl.BlockSpec((_NB, _D), lambda i: (i, 0)),
        out_shape=jax.ShapeDtypeStruct((batch, _D), jnp.float32),
    )


def kernel(indices, embeddings):
    b, h = indices.shape
    assert h == _HIST and embeddings.shape[1] == _D
    pool, nw = _make_pooler(b)
    idx_t = indices.astype(jnp.int32).T
    sums = pool(idx_t, embeddings.astype(jnp.float32))
    return _make_normalizer(b)(sums)


# speedup vs baseline: 1.5510x; 1.0433x over previous
import functools

import jax
import jax.numpy as jnp
from jax import lax
from jax.experimental import pallas as pl
from jax.experimental.pallas import tpu as pltpu
from jax.experimental.pallas import tpu_sc as plsc

_D = 64
_HIST = 50
_L = 16
_CB = 128
_NBUF = 4
_NB = 1024


def _sc_geometry():
    try:
        info = plsc.get_sparse_core_info()
        return info.num_cores, info.num_subcores
    except Exception:
        return 2, 16


@functools.lru_cache(maxsize=None)
def _make_pooler(batch):
    nc, ns = _sc_geometry()
    nw = nc * ns
    bw = batch // nw
    nchunk_b = bw // _CB
    nchunk = _HIST * nchunk_b
    mesh = plsc.VectorSubcoreMesh(core_axis_name="c", subcore_axis_name="s")

    @functools.partial(
        pl.kernel,
        mesh=mesh,
        out_type=jax.ShapeDtypeStruct((batch, _D), jnp.float32),
        scratch_types=[
            pltpu.VMEM((_HIST, bw), jnp.int32),
            pltpu.VMEM((bw, _D), jnp.float32),
            *[pltpu.SemaphoreType.DMA for _ in range(_NBUF)],
        ],
        compiler_params=pltpu.CompilerParams(use_tc_tiling_on_sc=False),
    )
    def pool(idx_hbm, table_hbm, out_hbm, idx_v, acc_v, *sems):
        w = lax.axis_index("s") * nc + lax.axis_index("c")
        base = w * bw
        pltpu.sync_copy(idx_hbm.at[:, pl.ds(base, bw)], idx_v)

        def idx_ref(c):
            j = c // nchunk_b
            b0 = (c % nchunk_b) * _CB
            return idx_v.at[j, pl.ds(b0, _CB)]

        def acc_ref(c):
            b0 = (c % nchunk_b) * _CB
            return acc_v.at[pl.ds(b0, _CB), :]

        def start(c, b, add=True):
            pltpu.async_copy(table_hbm.at[idx_ref(c)], acc_ref(c), sems[b], add=add)

        def drain(c, b):
            pltpu.make_async_copy(table_hbm.at[idx_ref(c)], acc_ref(c), sems[b]).wait()

        assert _NBUF == nchunk_b
        for b in range(_NBUF):
            start(b, b, add=False)

        def group(gi, carry):
            c0 = gi * _NBUF
            for b in range(_NBUF):
                c = c0 + b
                drain(c, b)

                @pl.when(c + _NBUF < nchunk)
                def _():
                    start(c + _NBUF, b)

            return carry

        lax.fori_loop(0, nchunk // _NBUF, group, 0)
        pltpu.sync_copy(acc_v, out_hbm.at[pl.ds(base, bw), :])

    return pool, nw


def _norm_kernel(x_ref, o_ref):
    x = x_ref[...]
    nsq = jnp.sum(x * x, axis=1, keepdims=True)
    clamp = jnp.float32((_HIST * 1e-12) ** 2)
    o_ref[...] = x * lax.rsqrt(jnp.maximum(nsq, clamp))


@functools.lru_cache(maxsize=None)
def _make_normalizer(batch):
    return pl.pallas_call(
        _norm_kernel,
        grid=(batch // _NB,),
        in_specs=[pl.BlockSpec((_NB, _D), lambda i: (i, 0))],
        out_specs=pl.BlockSpec((_NB, _D), lambda i: (i, 0)),
        out_shape=jax.ShapeDtypeStruct((batch, _D), jnp.float32),
    )


def kernel(indices, embeddings):
    b, h = indices.shape
    assert h == _HIST and embeddings.shape[1] == _D
    pool, nw = _make_pooler(b)
    idx_t = indices.astype(jnp.int32).T
    sums = pool(idx_t, embeddings.astype(jnp.float32))
    return _make_normalizer(b)(sums)
